# baseline (speedup 1.0000x reference)
import functools

import jax
import jax.numpy as jnp
from jax import lax
from jax.experimental import pallas as pl
from jax.experimental.pallas import tpu as pltpu
from jax.experimental.pallas import tpu_sc as plsc

_L = 16
_NS = 16
_C = 64


def _pad_rows(n: int) -> int:
  blk = _NS * 128
  return -(-n // blk) * blk


def _build_sc_kernel(np_e: int, np_r: int, n_edges: int, half: int):
  n_chunks = n_edges // _C
  chunks_per_tile = -(-n_chunks // _NS)
  n_slots = -(-(chunks_per_tile + 2) // 6) * 6
  rows_e = np_e // _NS
  rows_r = np_r // _NS
  assert n_edges % _C == 0 and half % _L == 0

  mesh = plsc.VectorSubcoreMesh(core_axis_name="c", subcore_axis_name="s")

  f32 = jnp.float32
  i32 = jnp.int32

  @functools.partial(
      pl.kernel,
      out_type=(
          jax.ShapeDtypeStruct((2 * np_e, half), f32),
          jax.ShapeDtypeStruct((2 * np_r, half), f32),
      ),
      mesh=mesh,
      compiler_params=pltpu.CompilerParams(use_tc_tiling_on_sc=False),
      scratch_types=(
          [pltpu.VMEM_SHARED((np_e, half), f32),
           pltpu.VMEM_SHARED((np_r, half), f32)]
          + [pltpu.VMEM((_C,), i32)] * 9
          + [pltpu.VMEM((_C,), i32)] * 3
          + [pltpu.VMEM((_C, half), f32)] * 10
          + [pltpu.SemaphoreType.DMA] * 4
      ),
  )
  def sc_kernel(e2, r2, hh, rr, tt, oe, out_r, acc_e, acc_r,
                ih0, ih1, ih2, ir0, ir1, ir2, it0, it1, it2,
                ioh, ior, iot,
                gh0, gh1, gr0, gr1, gt0, gt1, vb0, vb1, mb0, mb1,
                gsem0, gsem1, ssem0, ssem1):
    c = lax.axis_index("c")
    s = lax.axis_index("s")
    ihs, irs, its = (ih0, ih1, ih2), (ir0, ir1, ir2), (it0, it1, it2)
    ghs, grs, gts = (gh0, gh1), (gr0, gr1), (gt0, gt1)
    vbs, mbs = (vb0, vb1), (mb0, mb1)
    gsems, ssems = (gsem0, gsem1), (ssem0, ssem1)

    pltpu.sync_copy(e2.at[pl.ds(c * np_e + s * rows_e, rows_e)],
                    acc_e.at[pl.ds(s * rows_e, rows_e)])
    pltpu.sync_copy(r2.at[pl.ds(c * np_r + s * rows_r, rows_r)],
                    acc_r.at[pl.ds(s * rows_r, rows_r)])
    plsc.subcore_barrier()

    coff_e = c * np_e
    coff_r = c * np_r
    dummy = e2.at[pl.ds(0, _C)]

    def load_idx_and_gather(cid, pi, p):
      base = cid * _C
      pltpu.sync_copy(hh.at[pl.ds(base, _C)], ihs[pi])
      pltpu.sync_copy(rr.at[pl.ds(base, _C)], irs[pi])
      pltpu.sync_copy(tt.at[pl.ds(base, _C)], its[pi])
      for k in range(_C // _L):
        sl = pl.ds(k * _L, _L)
        ioh[sl] = ihs[pi][sl] + coff_e
        ior[sl] = irs[pi][sl] + coff_r
        iot[sl] = its[pi][sl] + coff_e
      pltpu.async_copy(e2.at[ioh], ghs[p], gsems[p])
      pltpu.async_copy(r2.at[ior], grs[p], gsems[p])
      pltpu.async_copy(e2.at[iot], gts[p], gsems[p])

    load_idx_and_gather(s, 0, 0)

    @pl.loop(0, n_slots // 6)
    def _outer(go):
      for b in range(6):
        p, pn, pi = b % 2, (b + 1) % 2, b % 3
        g = go * 6 + b
        cid = g * _NS + s

        @pl.when((g >= 2) & (cid - 2 * _NS < n_chunks))
        def _():
          for _ in range(3):
            pltpu.make_async_copy(dummy, vbs[p], ssems[p]).wait()

        @pl.when(cid < n_chunks)
        def _():
          pltpu.make_async_copy(dummy, ghs[p], gsems[p]).wait()
          pltpu.make_async_copy(dummy, grs[p], gsems[p]).wait()
          pltpu.make_async_copy(dummy, gts[p], gsems[p]).wait()

        @pl.when(cid + _NS < n_chunks)
        def _():
          load_idx_and_gather(cid + _NS, (b + 1) % 3, pn)

        @pl.when(cid < n_chunks)
        def _():
          gh, gr, gt, vb, mb = ghs[p], grs[p], gts[p], vbs[p], mbs[p]

          @pl.loop(0, _C, unroll=2)
          def _rows(row):
            for k in range(half // _L):
              sl = pl.ds(k * _L, _L)
              v = gh[row, sl] + gr[row, sl] - gt[row, sl]
              vb[row, sl] = v
              mb[row, sl] = -v

          pltpu.async_copy(mb, acc_e.at[ihs[pi]], ssems[p], add=True)
          pltpu.async_copy(vb, acc_e.at[its[pi]], ssems[p], add=True)
          pltpu.async_copy(mb, acc_r.at[irs[pi]], ssems[p], add=True)

    plsc.subcore_barrier()

    def relu_out(acc, out_ref, coff, rows):
      for blk in range(rows // _C):
        row0 = s * rows + blk * _C
        pltpu.sync_copy(acc.at[pl.ds(row0, _C)], gh0)

        @pl.loop(0, _C, unroll=2)
        def _rl(row):
          for k in range(half // _L):
            sl = pl.ds(k * _L, _L)
            gh0[row, sl] = jnp.maximum(gh0[row, sl], 0.0)

        pltpu.sync_copy(gh0, out_ref.at[pl.ds(coff + row0, _C)])

    relu_out(acc_e, oe, coff_e, rows_e)
    relu_out(acc_r, out_r, coff_r, rows_r)

  return sc_kernel


def kernel(ent_emb, rel_emb, nei_array):
  n_nodes, d = ent_emb.shape
  n_rels = rel_emb.shape[0]
  n_edges = nei_array.shape[0]
  half = d // 2
  np_e = _pad_rows(n_nodes)
  np_r = _pad_rows(n_rels)

  nei = nei_array.astype(jnp.int32)
  h_idx = nei[:, 0]
  r_idx = nei[:, 1]
  t_idx = nei[:, 2]

  def stack(tab, np_n):
    n = tab.shape[0]
    pad = jnp.zeros((np_n - n, half), jnp.float32)
    return jnp.concatenate([tab[:, :half], pad, tab[:, half:], pad], axis=0)

  e2 = stack(ent_emb, np_e)
  r2 = stack(rel_emb, np_r)

  oe2, or2 = _build_sc_kernel(np_e, np_r, n_edges, half)(
      e2, r2, h_idx, r_idx, t_idx)

  ent_out = jnp.concatenate([oe2[:n_nodes], oe2[np_e:np_e + n_nodes]], axis=1)
  rel_out = jnp.concatenate([or2[:n_rels], or2[np_r:np_r + n_rels]], axis=1)
  return ent_out, rel_out
